# baseline (speedup 1.0000x reference)
import functools

import jax
import jax.numpy as jnp
from jax import lax
from jax.experimental import pallas as pl
from jax.experimental.pallas import tpu as pltpu
from jax.experimental.pallas import tpu_sc as plsc

N = 100000
D = 128
WINDOW = 400
NWIN = N // WINDOW
NUM_CORES = 2
NUM_SUBCORES = 16
NW = NUM_CORES * NUM_SUBCORES
MAX_ITERS = -(-NWIN // NW)


def _sc_lookup(chi, emb_weight):
    mesh = plsc.VectorSubcoreMesh(core_axis_name="c", subcore_axis_name="s")

    @functools.partial(
        pl.kernel,
        mesh=mesh,
        out_type=jax.ShapeDtypeStruct((N, D), jnp.float32),
        scratch_types=[
            pltpu.VMEM_SHARED((3, D), jnp.float32),
            pltpu.VMEM((WINDOW,), jnp.int32),
            pltpu.VMEM((WINDOW,), jnp.int32),
            pltpu.VMEM((WINDOW, D), jnp.float32),
            pltpu.VMEM((WINDOW, D), jnp.float32),
            pltpu.SemaphoreType.DMA,
            pltpu.SemaphoreType.DMA,
            pltpu.SemaphoreType.DMA,
            pltpu.SemaphoreType.DMA,
        ],
    )
    def k(table_hbm, idx_hbm, out_hbm, table_sh, idx0, idx1, rows0, rows1,
          sem_g0, sem_g1, sem_w0, sem_w1):
        wid = lax.axis_index("s") * NUM_CORES + lax.axis_index("c")

        @pl.when(lax.axis_index("s") == 0)
        def _():
            pltpu.sync_copy(table_hbm, table_sh)

        plsc.subcore_barrier()

        sem_g = (sem_g0, sem_g1)
        sem_w = (sem_w0, sem_w1)
        idx_b = (idx0, idx1)
        rows_b = (rows0, rows1)
        gathers = [None] * MAX_ITERS
        writebacks = [None] * MAX_ITERS

        def start_gather(k_step):
            b = k_step % 2
            win = k_step * NW + wid
            base = win * WINDOW
            gathers[k_step] = pltpu.make_async_copy(
                table_sh.at[idx_b[b]], rows_b[b], sem_g[b])

            @pl.when(win < NWIN)
            def _():
                pltpu.sync_copy(idx_hbm.at[pl.ds(base, WINDOW)], idx_b[b])
                gathers[k_step].start()

        def finish_window(k_step):
            b = k_step % 2
            win = k_step * NW + wid
            base = win * WINDOW
            writebacks[k_step] = pltpu.make_async_copy(
                rows_b[b], out_hbm.at[pl.ds(base, WINDOW)], sem_w[b])

            @pl.when(win < NWIN)
            def _():
                gathers[k_step].wait()
                writebacks[k_step].start()

        def wait_writeback(k_step):
            b = k_step % 2
            win = k_step * NW + wid

            @pl.when(win < NWIN)
            def _():
                writebacks[k_step].wait()

        start_gather(0)
        for k_step in range(1, MAX_ITERS):
            if k_step >= 2:
                wait_writeback(k_step - 2)
            start_gather(k_step)
            finish_window(k_step - 1)
        wait_writeback(MAX_ITERS - 2)
        finish_window(MAX_ITERS - 1)
        wait_writeback(MAX_ITERS - 1)

    return k(emb_weight, chi)


def kernel(chi, emb_weight):
    chi = chi.astype(jnp.int32)
    emb_weight = emb_weight.astype(jnp.float32)
    return _sc_lookup(chi, emb_weight)


# speedup vs baseline: 1.0029x; 1.0029x over previous
import functools

import jax
import jax.numpy as jnp
from jax import lax
from jax.experimental import pallas as pl
from jax.experimental.pallas import tpu as pltpu
from jax.experimental.pallas import tpu_sc as plsc

N = 100000
D = 128
NUM_CORES = 2
NUM_SUBCORES = 16
NW = NUM_CORES * NUM_SUBCORES
SPAN = N // NW
CHUNK = 3128
WINDOW = 400
TAIL = 328
NFULL = 7
STEPS = NFULL + 1


def _sc_lookup(chi, emb_weight):
    mesh = plsc.VectorSubcoreMesh(core_axis_name="c", subcore_axis_name="s")

    @functools.partial(
        pl.kernel,
        mesh=mesh,
        out_type=jax.ShapeDtypeStruct((N, D), jnp.float32),
        scratch_types=[
            pltpu.VMEM_SHARED((3, D), jnp.float32),
            pltpu.VMEM((CHUNK,), jnp.int32),
            pltpu.VMEM((WINDOW, D), jnp.float32),
            pltpu.VMEM((WINDOW, D), jnp.float32),
            pltpu.SemaphoreType.DMA,
            pltpu.SemaphoreType.DMA,
            pltpu.SemaphoreType.DMA,
            pltpu.SemaphoreType.DMA,
        ],
    )
    def k(table_hbm, idx_hbm, out_hbm, table_sh, idx_c, rows0, rows1,
          sem_g0, sem_g1, sem_w0, sem_w1):
        wid = lax.axis_index("s") * NUM_CORES + lax.axis_index("c")
        base = (SPAN * wid) // 8 * 8
        next_base = (SPAN * (wid + 1)) // 8 * 8
        tail_off = (next_base - base) - TAIL

        @pl.when(lax.axis_index("s") == 0)
        def _():
            pltpu.sync_copy(table_hbm, table_sh)

        plsc.subcore_barrier()
        pltpu.sync_copy(idx_hbm.at[pl.ds(base, CHUNK)], idx_c)

        sem_g = (sem_g0, sem_g1)
        sem_w = (sem_w0, sem_w1)
        rows_b = (rows0, rows1)
        gathers = [None] * STEPS
        writebacks = [None] * STEPS

        def start_gather(k_step):
            b = k_step % 2
            if k_step < NFULL:
                src = table_sh.at[idx_c.at[pl.ds(k_step * WINDOW, WINDOW)]]
                dst = rows_b[b]
            else:
                src = table_sh.at[idx_c.at[pl.ds(tail_off, TAIL)]]
                dst = rows_b[b].at[pl.ds(0, TAIL)]
            gathers[k_step] = pltpu.make_async_copy(src, dst, sem_g[b])
            gathers[k_step].start()

        def finish_window(k_step):
            b = k_step % 2
            if k_step < NFULL:
                src = rows_b[b]
                dst = out_hbm.at[pl.ds(base + k_step * WINDOW, WINDOW)]
            else:
                src = rows_b[b].at[pl.ds(0, TAIL)]
                dst = out_hbm.at[pl.ds(base + tail_off, TAIL)]
            gathers[k_step].wait()
            writebacks[k_step] = pltpu.make_async_copy(src, dst, sem_w[b])
            writebacks[k_step].start()

        start_gather(0)
        for k_step in range(1, STEPS):
            if k_step >= 2:
                writebacks[k_step - 2].wait()
            start_gather(k_step)
            finish_window(k_step - 1)
        writebacks[STEPS - 2].wait()
        finish_window(STEPS - 1)
        writebacks[STEPS - 1].wait()

    return k(emb_weight, chi)


def kernel(chi, emb_weight):
    chi = chi.astype(jnp.int32)
    emb_weight = emb_weight.astype(jnp.float32)
    return _sc_lookup(chi, emb_weight)


# speedup vs baseline: 1.0304x; 1.0274x over previous
import functools

import jax
import jax.numpy as jnp
from jax import lax
from jax.experimental import pallas as pl
from jax.experimental.pallas import tpu as pltpu
from jax.experimental.pallas import tpu_sc as plsc

N = 100000
D = 128
WINDOW = 400
NWIN = N // WINDOW


def _sc_lookup(chi, emb_weight):
    mesh = plsc.VectorSubcoreMesh(core_axis_name="c", subcore_axis_name="s")
    chi3d = chi.reshape(NWIN, 1, WINDOW)

    @functools.partial(
        pl.kernel,
        mesh=mesh,
        out_type=jax.ShapeDtypeStruct((N, D), jnp.float32),
        scratch_types=[
            pltpu.VMEM_SHARED((3, D), jnp.float32),
        ],
    )
    def k(table_hbm, idx_hbm, out_hbm, table_sh):
        @pl.when(lax.axis_index("s") == 0)
        def _():
            pltpu.sync_copy(table_hbm, table_sh)

        plsc.subcore_barrier()

        def body(i_vmem, o_vmem):
            pltpu.sync_copy(table_sh.at[i_vmem.at[0, 0]], o_vmem)

        pltpu.emit_pipeline(
            body,
            grid=(NWIN,),
            in_specs=[pl.BlockSpec((1, 1, WINDOW), index_map=lambda i: (i, 0, 0))],
            out_specs=[pl.BlockSpec((WINDOW, D), index_map=lambda i: (i, 0))],
            core_axis_name=("c", "s"),
            dimension_semantics=(pltpu.PARALLEL,),
        )(idx_hbm, out_hbm)

    return k(emb_weight, chi3d)


def kernel(chi, emb_weight):
    chi = chi.astype(jnp.int32)
    emb_weight = emb_weight.astype(jnp.float32)
    return _sc_lookup(chi, emb_weight)
